# baseline (speedup 1.0000x reference)
import functools

import jax
import jax.numpy as jnp
from jax import lax
from jax.experimental import pallas as pl
from jax.experimental.pallas import tpu as pltpu
from jax.experimental.pallas import tpu_sc as plsc

N_TYPES = 8
BM = 256
NB = 8
NTILES = NB + N_TYPES - 1
L = 16


def _lane(vec, e):
    return jnp.sum(jnp.where(jnp.arange(L, dtype=jnp.int32) == e, vec, 0))


def _sc_route(st):
    n = st.shape[0]
    ns = 16
    ch = n // ns
    mesh = plsc.VectorSubcoreMesh(
        core_axis_name="c", subcore_axis_name="s", num_cores=1,
        num_subcores=ns)

    @functools.partial(
        pl.kernel, mesh=mesh,
        out_type=(jax.ShapeDtypeStruct((n,), jnp.int32),
                  jax.ShapeDtypeStruct((3 * L,), jnp.int32)),
        compiler_params=pltpu.CompilerParams(needs_layout_passes=False),
        scratch_types=[
            pltpu.VMEM((ch,), jnp.int32),
            pltpu.VMEM((ch,), jnp.int32),
            pltpu.VMEM((ch,), jnp.int32),
            pltpu.VMEM((L,), jnp.int32),
            pltpu.VMEM((ns * L,), jnp.int32),
            pltpu.VMEM((L,), jnp.int32),
            pltpu.VMEM_SHARED((ns * L,), jnp.int32),
        ],
    )
    def k(st_hbm, rank_hbm, meta_hbm, st_v, lp_v, idx_v, cnts_v, all_v,
          row_v, shared):
        sid = lax.axis_index("s")
        base = sid * ch
        iota = jnp.arange(L, dtype=jnp.int32)
        pltpu.sync_copy(st_hbm.at[pl.ds(base, ch)], st_v)

        carry = [jnp.int32(0)] * N_TYPES
        for j in range(ch // L):
            c = st_v[pl.ds(j * L, L)]
            r = jnp.zeros((L,), jnp.int32)
            for e in range(N_TYPES):
                me = c == e
                mi = me.astype(jnp.int32)
                cs = plsc.cumsum(mi)
                r = jnp.where(me, carry[e] + cs - 1, r)
                carry[e] = carry[e] + jnp.sum(mi)
            lp_v[pl.ds(j * L, L)] = r
        cnt_vec = jnp.zeros((L,), jnp.int32)
        for e in range(N_TYPES):
            cnt_vec = jnp.where(iota == e, carry[e], cnt_vec)
        cnts_v[...] = cnt_vec

        pltpu.sync_copy(cnts_v, shared.at[pl.ds(base // ch * L, L)])
        plsc.subcore_barrier()
        pltpu.sync_copy(shared, all_v)
        total = jnp.zeros((L,), jnp.int32)
        before = jnp.zeros((L,), jnp.int32)
        for w in range(ns):
            rv = all_v[pl.ds(w * L, L)]
            total = total + rv
            before = before + rv * (w < sid).astype(jnp.int32)
        incl = plsc.cumsum(total)
        offs = incl - total
        base_vec = offs + before

        base_sc = [_lane(base_vec, e) for e in range(N_TYPES)]
        for j in range(ch // L):
            c = st_v[pl.ds(j * L, L)]
            radd = jnp.zeros((L,), jnp.int32)
            for e in range(N_TYPES):
                radd = jnp.where(c == e, base_sc[e], radd)
            idx_v[pl.ds(j * L, L)] = radd + lp_v[pl.ds(j * L, L)]
        pltpu.sync_copy(idx_v, rank_hbm.at[pl.ds(base, ch)])

        @pl.when(sid == 0)
        def _meta():
            row_v[...] = offs
            pltpu.sync_copy(row_v, meta_hbm.at[pl.ds(0, L)])
            lo_sc = [_lane(offs, e) for e in range(N_TYPES)]
            hi_sc = [_lane(incl, e) for e in range(N_TYPES)]
            tv = jnp.full((L,), N_TYPES * NB - 1, jnp.int32)
            carry3 = jnp.int32(0)
            for q in range(N_TYPES * NB // L):
                pid = q * L + iota
                pe = pid // NB
                pb = pid % NB
                lo = jnp.zeros((L,), jnp.int32)
                hi = jnp.zeros((L,), jnp.int32)
                for e in range(N_TYPES):
                    lo = jnp.where(pe == e, lo_sc[e], lo)
                    hi = jnp.where(pe == e, hi_sc[e], hi)
                ex = (lo < (pb + 1) * BM) & (hi > pb * BM) & (hi > lo)
                exi = ex.astype(jnp.int32)
                pos = carry3 + plsc.cumsum(exi) - 1
                for l in range(L):
                    tv = jnp.where(
                        (iota == _lane(pos, l)) & (_lane(exi, l) == 1),
                        q * L + l, tv)
                carry3 = carry3 + jnp.sum(exi)
            lastpid = jnp.sum(jnp.where(iota == carry3 - 1, tv, 0))
            tv = jnp.where(iota >= carry3, lastpid, tv)
            row_v[...] = tv
            pltpu.sync_copy(row_v, meta_hbm.at[pl.ds(L, L)])
            row_v[...] = jnp.where(iota == 0, carry3, 0)
            pltpu.sync_copy(row_v, meta_hbm.at[pl.ds(2 * L, L)])

    return k(st)


def _sc_scatter_rows(vals, idx, n_out):
    B, D = vals.shape
    info = plsc.get_sparse_core_info()
    NC, NS = info.num_cores, info.num_subcores
    b_per_w = B // (NC * NS)
    mesh = plsc.VectorSubcoreMesh(core_axis_name="c", subcore_axis_name="s")

    @functools.partial(
        pl.kernel, mesh=mesh,
        out_type=jax.ShapeDtypeStruct((n_out, D), jnp.float32),
        scratch_types=[
            pltpu.VMEM((b_per_w,), jnp.int32),
            pltpu.VMEM((b_per_w, D), jnp.float32),
            pltpu.SemaphoreType.DMA,
        ],
    )
    def k(vals_hbm, idx_hbm, out_hbm, idx_v, rows_v, sem):
        wid = lax.axis_index("s") * NC + lax.axis_index("c")
        base = wid * b_per_w
        pltpu.sync_copy(idx_hbm.at[pl.ds(base, b_per_w)], idx_v)
        pltpu.sync_copy(vals_hbm.at[pl.ds(base, b_per_w)], rows_v)
        pltpu.async_copy(rows_v, out_hbm.at[idx_v], sem).wait()

    return k(vals, idx)


def _sc_gather_rows(table, idx):
    V, D = table.shape
    B = idx.shape[0]
    info = plsc.get_sparse_core_info()
    NC, NS = info.num_cores, info.num_subcores
    b_per_w = B // (NC * NS)
    mesh = plsc.VectorSubcoreMesh(core_axis_name="c", subcore_axis_name="s")

    @functools.partial(
        pl.kernel, mesh=mesh,
        out_type=jax.ShapeDtypeStruct((B, D), jnp.float32),
        scratch_types=[
            pltpu.VMEM((b_per_w,), jnp.int32),
            pltpu.VMEM((b_per_w, D), jnp.float32),
            pltpu.SemaphoreType.DMA,
        ],
    )
    def k(table_hbm, idx_hbm, out_hbm, idx_v, rows_v, sem):
        wid = lax.axis_index("s") * NC + lax.axis_index("c")
        base = wid * b_per_w
        pltpu.sync_copy(idx_hbm.at[pl.ds(base, b_per_w)], idx_v)
        pltpu.async_copy(table_hbm.at[idx_v], rows_v, sem).wait()
        pltpu.sync_copy(rows_v, out_hbm.at[pl.ds(base, b_per_w)])

    return k(table, idx)


def _grouped_body(meta_ref, x_ref, wp_ref, bp_ref, wt_ref, bt_ref,
                  wc1_ref, bc1_ref, wc2t_ref, bc2_ref,
                  y_ref, h1_scr, acc_scr):
    t = pl.program_id(0)
    pid = meta_ref[16 + t]
    e = pid // NB
    b = pid % NB
    nt = meta_ref[32]
    valid = t < nt
    prev_b = meta_ref[16 + jnp.maximum(t - 1, 0)] % NB
    next_b = meta_ref[16 + t + 1] % NB
    isf = ((t == 0) | (prev_b != b)) & valid
    isl = ((t == nt - 1) | (next_b != b)) & valid

    @pl.when(isf)
    def _init():
        xb = x_ref[:]
        h1_scr[:] = jnp.maximum(
            jnp.dot(xb.astype(jnp.bfloat16), wp_ref[:],
                    preferred_element_type=jnp.float32)
            + bp_ref[:], 0.0)
        acc_scr[:] = xb

    r = b * BM + lax.broadcasted_iota(jnp.int32, (BM, 1), 0)
    seg = (r >= meta_ref[e]) & (r < meta_ref[e + 1]) & valid
    oe = jnp.maximum(
        jnp.dot(h1_scr[:].astype(jnp.bfloat16), wt_ref[0],
                preferred_element_type=jnp.float32)
        + bt_ref[pl.ds(e, 1), :], 0.0)
    acc_scr[:] = acc_scr[:] + jnp.where(seg, oe, 0.0)

    @pl.when(isl)
    def _contribs():
        h2 = jnp.maximum(
            jnp.dot(acc_scr[:].astype(jnp.bfloat16), wc1_ref[:],
                    preferred_element_type=jnp.float32)
            + bc1_ref[:], 0.0)
        yv = (jnp.sum(h2 * wc2t_ref[:], axis=1, keepdims=True)
              + bc2_ref[:])
        y_ref[:] = jnp.broadcast_to(yv, (BM, 128))


def kernel(x, sc_types, W_pre, b_pre, W_type, b_type, W_c1, b_c1, W_c2, b_c2):
    d = x.shape[-1]
    xf = x.reshape(-1, d)
    n = xf.shape[0]
    st = sc_types.reshape(-1).astype(jnp.int32)

    rank, meta = _sc_route(st)
    x_s = _sc_scatter_rows(xf, rank, n)

    grid_spec = pltpu.PrefetchScalarGridSpec(
        num_scalar_prefetch=1,
        grid=(NTILES,),
        in_specs=[
            pl.BlockSpec((BM, d), lambda t, m: (m[16 + t] % NB, 0)),
            pl.BlockSpec(W_pre.shape, lambda t, m: (0, 0)),
            pl.BlockSpec((1, d), lambda t, m: (0, 0)),
            pl.BlockSpec((1, d, d), lambda t, m: (m[16 + t] // NB, 0, 0)),
            pl.BlockSpec(b_type.shape, lambda t, m: (0, 0)),
            pl.BlockSpec(W_c1.shape, lambda t, m: (0, 0)),
            pl.BlockSpec((1, d), lambda t, m: (0, 0)),
            pl.BlockSpec((1, d), lambda t, m: (0, 0)),
            pl.BlockSpec((1, 1), lambda t, m: (0, 0)),
        ],
        out_specs=pl.BlockSpec((BM, 128), lambda t, m: (m[16 + t] % NB, 0)),
        scratch_shapes=[
            pltpu.VMEM((BM, d), jnp.float32),
            pltpu.VMEM((BM, d), jnp.float32),
        ],
    )
    y_s = pl.pallas_call(
        _grouped_body,
        grid_spec=grid_spec,
        out_shape=jax.ShapeDtypeStruct((n, 128), jnp.float32),
    )(meta,
      x_s, W_pre.astype(jnp.bfloat16), b_pre.reshape(1, -1),
      W_type.astype(jnp.bfloat16), b_type,
      W_c1.astype(jnp.bfloat16), b_c1.reshape(1, -1),
      W_c2.reshape(1, -1), b_c2.reshape(1, 1))

    return _sc_gather_rows(y_s, rank)[:, :1]


# speedup vs baseline: 1.5259x; 1.5259x over previous
import jax
import jax.numpy as jnp
from jax.experimental import pallas as pl

N_TYPES = 8
BM = 256


def _fused_body(st_ref, x_ref, wp_ref, bp_ref, wt_ref, bt_ref,
                wc1_ref, bc1_ref, wc2t_ref, bc2_ref, y_ref):
    x = x_ref[:]
    st = st_ref[:]
    h1 = jnp.maximum(
        jnp.dot(x, wp_ref[:], preferred_element_type=jnp.float32)
        + bp_ref[:], 0.0)
    acc = jnp.zeros_like(x)
    for e in range(N_TYPES):
        oe = jnp.maximum(
            jnp.dot(h1, wt_ref[e], preferred_element_type=jnp.float32)
            + bt_ref[e:e + 1, :], 0.0)
        acc = acc + jnp.where(st == e, oe, 0.0)
    xo = x + acc
    h2 = jnp.maximum(
        jnp.dot(xo, wc1_ref[:], preferred_element_type=jnp.float32)
        + bc1_ref[:], 0.0)
    y = jnp.sum(h2 * wc2t_ref[:], axis=1, keepdims=True) + bc2_ref[:]
    y_ref[:] = y


def kernel(x, sc_types, W_pre, b_pre, W_type, b_type, W_c1, b_c1, W_c2, b_c2):
    d = x.shape[-1]
    xf = x.reshape(-1, d)
    n = xf.shape[0]
    st = sc_types.reshape(-1, 1).astype(jnp.int32)
    nb = n // BM

    grid = (nb,)
    y = pl.pallas_call(
        _fused_body,
        grid=grid,
        in_specs=[
            pl.BlockSpec((BM, 1), lambda i: (i, 0)),
            pl.BlockSpec((BM, d), lambda i: (i, 0)),
            pl.BlockSpec(W_pre.shape, lambda i: (0, 0)),
            pl.BlockSpec((1, d), lambda i: (0, 0)),
            pl.BlockSpec(W_type.shape, lambda i: (0, 0, 0)),
            pl.BlockSpec(b_type.shape, lambda i: (0, 0)),
            pl.BlockSpec(W_c1.shape, lambda i: (0, 0)),
            pl.BlockSpec((1, d), lambda i: (0, 0)),
            pl.BlockSpec((1, d), lambda i: (0, 0)),
            pl.BlockSpec((1, 1), lambda i: (0, 0)),
        ],
        out_specs=pl.BlockSpec((BM, 1), lambda i: (i, 0)),
        out_shape=jax.ShapeDtypeStruct((n, 1), jnp.float32),
    )(st, xf, W_pre, b_pre.reshape(1, -1), W_type, b_type,
      W_c1, b_c1.reshape(1, -1), W_c2.reshape(1, -1), b_c2.reshape(1, 1))
    return y
